# baseline (device time: 105979 ns/iter reference)
import jax
import jax.numpy as jnp
from jax import lax
from jax.experimental import pallas as pl
from jax.experimental.pallas import tpu as pltpu

N_DEV = 4
E_LOC = 4
N_EXP = 16
CAP = 409


def _moe_body(x_ref, w_ref, oh_ref, lr_ref, cnt_ref, pm_ref, out_ref,
              wbuf, cbuf, send_sems, recv_sems):
    my = lax.axis_index("i")
    left = (my - 1) % N_DEV
    right = (my + 1) % N_DEV
    opp = (my + 2) % N_DEV
    hdim = out_ref.shape[-1]
    hh = hdim // 2

    barrier = pltpu.get_barrier_semaphore()
    for nbr in (left, right, opp):
        pl.semaphore_signal(
            barrier, inc=1, device_id=(nbr,),
            device_id_type=pl.DeviceIdType.MESH,
        )
    pl.semaphore_wait(barrier, N_DEV - 1)

    def msg(k, src, dst, dev):
        return pltpu.make_async_remote_copy(
            src_ref=src, dst_ref=dst,
            send_sem=send_sems.at[k], recv_sem=recv_sems.at[k],
            device_id=(dev,), device_id_type=pl.DeviceIdType.MESH,
        )

    c6 = msg(6, cnt_ref, cbuf.at[1:2], right)
    c7 = msg(7, cnt_ref, cbuf.at[2:3], left)
    c8 = msg(8, cnt_ref, cbuf.at[3:4], opp)
    c6.start()
    c7.start()
    c8.start()

    m0 = msg(0, w_ref.at[0:2], wbuf.at[0, 0:2], right)
    m1 = msg(1, w_ref.at[2:4], wbuf.at[0, 2:4], right)
    m2 = msg(2, w_ref.at[0:2], wbuf.at[1, 0:2], left)
    m3 = msg(3, w_ref.at[2:4], wbuf.at[1, 2:4], left)
    m4 = msg(4, wbuf.at[0, 0:2], wbuf.at[2, 0:2], right)
    m5 = msg(5, wbuf.at[1, 2:4], wbuf.at[2, 2:4], left)
    m0.start()
    m1.start()
    m3.start()
    m2.start()

    c6.wait_recv()
    c7.wait_recv()
    c8.wait_recv()
    pref = jnp.dot(cnt_ref[...], pm_ref[0],
                   preferred_element_type=jnp.float32)
    for r in range(1, N_DEV):
        pref += jnp.dot(cbuf[r:r + 1, :], pm_ref[r],
                        preferred_element_type=jnp.float32)
    gate = oh_ref[...] * ((lr_ref[...] + pref) < CAP).astype(jnp.bfloat16)

    x = x_ref[...].astype(jnp.bfloat16)
    d = x.shape[1]

    def xg_of(slot):
        return jnp.concatenate(
            [x * gate[:, slot * E_LOC + le:slot * E_LOC + le + 1]
             for le in range(E_LOC)], axis=1)

    def chunk_out(w_chunk, slot):
        wmat = w_chunk.reshape(E_LOC * d, w_chunk.shape[-1])
        return jnp.dot(xg_of(slot), wmat, preferred_element_type=jnp.float32)

    out_ref[...] = chunk_out(w_ref[...], 0)

    m0.wait_recv()
    m4.start()
    m3.wait_recv()
    m5.start()

    m1.wait_recv()
    out_ref[...] += chunk_out(wbuf[0], 1)
    m2.wait_recv()
    out_ref[...] += chunk_out(wbuf[1], 2)
    m4.wait_recv()
    m5.wait_recv()
    out_ref[...] += chunk_out(wbuf[2], 3)

    for m in (m0, m1, m2, m3, m4, m5, c6, c7, c8):
        m.wait_send()


def _moe_call(x, w_bf, oh_cols, lr_cols, counts_row, pmats,
              n_tok, d, hdim):
    return pl.pallas_call(
        _moe_body,
        out_shape=jax.ShapeDtypeStruct((n_tok, hdim), jnp.float32),
        in_specs=[pl.BlockSpec(memory_space=pltpu.VMEM)] * 6,
        out_specs=pl.BlockSpec(memory_space=pltpu.VMEM),
        scratch_shapes=[
            pltpu.VMEM((N_DEV - 1, E_LOC, d, hdim), jnp.bfloat16),
            pltpu.VMEM((N_DEV, N_EXP), jnp.float32),
            pltpu.SemaphoreType.DMA((9,)),
            pltpu.SemaphoreType.DMA((9,)),
        ],
        compiler_params=pltpu.CompilerParams(
            collective_id=0, vmem_limit_bytes=48 * 1024 * 1024),
    )(x, w_bf, oh_cols, lr_cols, counts_row, pmats)


def kernel(x, router_W, route_idx, expert_W):
    del router_W
    n_tok, d = x.shape
    hdim = expert_W.shape[-1]
    p = lax.axis_index("i")

    route = route_idx[:, 0]
    e_ids = jnp.arange(N_EXP, dtype=route.dtype)
    counts = jnp.sum(route[:, None] == e_ids[None, :], axis=0)
    counts_row = counts.astype(jnp.float32).reshape(1, N_EXP)

    orig = (p + jnp.array([0, -1, 1, 2], jnp.int32)) % N_DEV
    cols = (orig[:, None] * E_LOC
            + jnp.arange(E_LOC, dtype=jnp.int32)[None, :]).reshape(-1)
    pmats = ((e_ids[None, :, None] == cols[None, None, :])
             & (orig[:, None, None] < p)
             ).astype(jnp.float32)

    oh_f = (route[:, None] == cols[None, :]).astype(jnp.float32)
    n_blk, blk = 16, n_tok // 16
    a = oh_f.reshape(n_blk, blk, N_EXP)
    tri = jnp.tril(jnp.ones((blk, blk), jnp.float32))
    within = jnp.matmul(tri[None], a)
    bsums = a.sum(axis=1)
    tri_s = jnp.tril(jnp.ones((n_blk, n_blk), jnp.float32), k=-1)
    offs = jnp.matmul(tri_s, bsums)
    lr_cols = (within + offs[:, None, :]).reshape(n_tok, N_EXP) - oh_f
    oh_cols = oh_f.astype(jnp.bfloat16)

    w_bf = expert_W.astype(jnp.bfloat16)
    return _moe_call(x, w_bf, oh_cols, lr_cols, counts_row, pmats,
                     n_tok, d, hdim)


# device time: 101162 ns/iter; 1.0476x vs baseline; 1.0476x over previous
import jax
import jax.numpy as jnp
from jax import lax
from jax.experimental import pallas as pl
from jax.experimental.pallas import tpu as pltpu

N_DEV = 4
E_LOC = 4
N_EXP = 16
CAP = 409


def _moe_body(x_ref, w_ref, oh_ref, lr_ref, cnt_ref, pm_ref, out_ref,
              wbuf, cbuf, acc, send_sems, recv_sems, out_sems):
    my = lax.axis_index("i")
    left = (my - 1) % N_DEV
    right = (my + 1) % N_DEV
    opp = (my + 2) % N_DEV
    hdim = out_ref.shape[-1]
    hh = hdim // 2

    barrier = pltpu.get_barrier_semaphore()
    for nbr in (left, right, opp):
        pl.semaphore_signal(
            barrier, inc=1, device_id=(nbr,),
            device_id_type=pl.DeviceIdType.MESH,
        )
    pl.semaphore_wait(barrier, N_DEV - 1)

    def msg(k, src, dst, dev):
        return pltpu.make_async_remote_copy(
            src_ref=src, dst_ref=dst,
            send_sem=send_sems.at[k], recv_sem=recv_sems.at[k],
            device_id=(dev,), device_id_type=pl.DeviceIdType.MESH,
        )

    c6 = msg(6, cnt_ref, cbuf.at[1:2], right)
    c7 = msg(7, cnt_ref, cbuf.at[2:3], left)
    c8 = msg(8, cnt_ref, cbuf.at[3:4], opp)
    c6.start()
    c7.start()
    c8.start()

    m0 = msg(0, w_ref.at[0:2], wbuf.at[0, 0:2], right)
    m1 = msg(1, w_ref.at[2:4], wbuf.at[0, 2:4], right)
    m2 = msg(2, w_ref.at[0:2], wbuf.at[1, 0:2], left)
    m3 = msg(3, w_ref.at[2:4], wbuf.at[1, 2:4], left)
    m4a = msg(4, wbuf.at[0, 0:2, :, 0:hh], wbuf.at[2, 0:2, :, 0:hh], right)
    m4b = msg(9, wbuf.at[0, 0:2, :, hh:], wbuf.at[2, 0:2, :, hh:], right)
    m5a = msg(5, wbuf.at[1, 2:4, :, 0:hh], wbuf.at[2, 2:4, :, 0:hh], left)
    m5b = msg(10, wbuf.at[1, 2:4, :, hh:], wbuf.at[2, 2:4, :, hh:], left)
    m0.start()
    m1.start()
    m3.start()
    m2.start()

    c6.wait_recv()
    c7.wait_recv()
    c8.wait_recv()
    pref = jnp.dot(cnt_ref[...], pm_ref[0],
                   preferred_element_type=jnp.float32)
    for r in range(1, N_DEV):
        pref += jnp.dot(cbuf[r:r + 1, :], pm_ref[r],
                        preferred_element_type=jnp.float32)
    gate = oh_ref[...] * ((lr_ref[...] + pref) < CAP).astype(jnp.bfloat16)

    x = x_ref[...].astype(jnp.bfloat16)
    d = x.shape[1]

    def xg_of(slot):
        return jnp.concatenate(
            [x * gate[:, slot * E_LOC + le:slot * E_LOC + le + 1]
             for le in range(E_LOC)], axis=1)

    def chunk_out(w_chunk, slot):
        wmat = w_chunk.reshape(E_LOC * d, w_chunk.shape[-1])
        return jnp.dot(xg_of(slot), wmat, preferred_element_type=jnp.float32)

    acc[...] = chunk_out(w_ref[...], 0)

    m0.wait_recv()
    m4a.start()
    m4b.start()
    m3.wait_recv()
    m5a.start()
    m5b.start()

    m1.wait_recv()
    acc[...] += chunk_out(wbuf[0], 1)
    m2.wait_recv()
    acc[...] += chunk_out(wbuf[1], 2)

    xg3 = xg_of(3)
    m4a.wait_recv()
    m5a.wait_recv()
    w3a = wbuf[2, :, :, 0:hh].reshape(E_LOC * d, hh)
    acc[:, 0:hh] += jnp.dot(xg3, w3a, preferred_element_type=jnp.float32)
    cp_a = pltpu.make_async_copy(acc.at[:, 0:hh], out_ref.at[:, 0:hh],
                                 out_sems.at[0])
    cp_a.start()
    m4b.wait_recv()
    m5b.wait_recv()
    w3b = wbuf[2, :, :, hh:].reshape(E_LOC * d, hdim - hh)
    acc[:, hh:] += jnp.dot(xg3, w3b, preferred_element_type=jnp.float32)
    cp_b = pltpu.make_async_copy(acc.at[:, hh:], out_ref.at[:, hh:],
                                 out_sems.at[1])
    cp_b.start()
    cp_a.wait()
    cp_b.wait()

    for m in (m0, m1, m2, m3, m4a, m4b, m5a, m5b, c6, c7, c8):
        m.wait_send()


def _moe_call(x, w_bf, oh_cols, lr_cols, counts_row, pmats,
              n_tok, d, hdim):
    return pl.pallas_call(
        _moe_body,
        out_shape=jax.ShapeDtypeStruct((n_tok, hdim), jnp.float32),
        in_specs=[pl.BlockSpec(memory_space=pltpu.VMEM)] * 6,
        out_specs=pl.BlockSpec(memory_space=pl.ANY),
        scratch_shapes=[
            pltpu.VMEM((N_DEV - 1, E_LOC, d, hdim), jnp.bfloat16),
            pltpu.VMEM((N_DEV, N_EXP), jnp.float32),
            pltpu.VMEM((n_tok, hdim), jnp.float32),
            pltpu.SemaphoreType.DMA((11,)),
            pltpu.SemaphoreType.DMA((11,)),
            pltpu.SemaphoreType.DMA((2,)),
        ],
        compiler_params=pltpu.CompilerParams(
            collective_id=0, vmem_limit_bytes=48 * 1024 * 1024),
    )(x, w_bf, oh_cols, lr_cols, counts_row, pmats)


def kernel(x, router_W, route_idx, expert_W):
    del router_W
    n_tok, d = x.shape
    hdim = expert_W.shape[-1]
    p = lax.axis_index("i")

    route = route_idx[:, 0]
    e_ids = jnp.arange(N_EXP, dtype=route.dtype)
    counts = jnp.sum(route[:, None] == e_ids[None, :], axis=0)
    counts_row = counts.astype(jnp.float32).reshape(1, N_EXP)

    orig = (p + jnp.array([0, -1, 1, 2], jnp.int32)) % N_DEV
    cols = (orig[:, None] * E_LOC
            + jnp.arange(E_LOC, dtype=jnp.int32)[None, :]).reshape(-1)
    pmats = ((e_ids[None, :, None] == cols[None, None, :])
             & (orig[:, None, None] < p)
             ).astype(jnp.float32)

    oh_f = (route[:, None] == cols[None, :]).astype(jnp.float32)
    n_blk, blk = 16, n_tok // 16
    a = oh_f.reshape(n_blk, blk, N_EXP)
    tri = jnp.tril(jnp.ones((blk, blk), jnp.float32))
    within = jnp.matmul(tri[None], a)
    bsums = a.sum(axis=1)
    tri_s = jnp.tril(jnp.ones((n_blk, n_blk), jnp.float32), k=-1)
    offs = jnp.matmul(tri_s, bsums)
    lr_cols = (within + offs[:, None, :]).reshape(n_tok, N_EXP) - oh_f
    oh_cols = oh_f.astype(jnp.bfloat16)

    w_bf = expert_W.astype(jnp.bfloat16)
    return _moe_call(x, w_bf, oh_cols, lr_cols, counts_row, pmats,
                     n_tok, d, hdim)
